# baseline (speedup 1.0000x reference)
import functools

import jax
import jax.numpy as jnp
from jax import lax
from jax.experimental import pallas as pl
from jax.experimental.pallas import tpu as pltpu
from jax.experimental.pallas import tpu_sc as plsc

N_NODES = 10000
N_EDGES = 320000
D = 128

NC = 2
NS = 16
LANES = 16

CHUNK = 64
UNROLL = 6
CHUNKS_PER_TILE = 168
E_PAD = NC * NS * CHUNKS_PER_TILE * CHUNK
STRIPE = 624
TAIL = N_NODES - NS * STRIPE


def _matmul(x, W):
    def body(x_ref, w_ref, o_ref):
        o_ref[...] = jnp.dot(x_ref[...], w_ref[...],
                             preferred_element_type=jnp.float32)

    blk = 1000
    return pl.pallas_call(
        body,
        grid=(N_NODES // blk,),
        in_specs=[
            pl.BlockSpec((blk, D), lambda i: (i, 0)),
            pl.BlockSpec((D, D), lambda i: (0, 0)),
        ],
        out_specs=pl.BlockSpec((blk, D), lambda i: (i, 0)),
        out_shape=jax.ShapeDtypeStruct((N_NODES, D), jnp.float32),
    )(x, W)


def _combine(partials, b2):
    def body(p_ref, b_ref, o_ref):
        o_ref[...] = p_ref[0] + p_ref[1] + b_ref[...]

    blk = 1000
    return pl.pallas_call(
        body,
        grid=(N_NODES // blk,),
        in_specs=[
            pl.BlockSpec((2, blk, D), lambda i: (0, i, 0)),
            pl.BlockSpec((1, D), lambda i: (0, 0)),
        ],
        out_specs=pl.BlockSpec((blk, D), lambda i: (i, 0)),
        out_shape=jax.ShapeDtypeStruct((N_NODES, D), jnp.float32),
    )(partials, b2)


def _sc_spmm(src1, dst1, w1, support):
    mesh = plsc.VectorSubcoreMesh(core_axis_name="c", subcore_axis_name="s")

    @functools.partial(
        pl.kernel,
        out_type=jax.ShapeDtypeStruct((NC, N_NODES, D), jnp.float32),
        mesh=mesh,
        scratch_types=[
            pltpu.VMEM((6, CHUNK), jnp.int32),
            pltpu.VMEM((6, CHUNK), jnp.int32),
            pltpu.VMEM((6, CHUNK), jnp.float32),
            pltpu.VMEM((CHUNK, D), jnp.float32),
            pltpu.VMEM((CHUNK, D), jnp.float32),
            pltpu.VMEM((CHUNK, D), jnp.float32),
            pltpu.VMEM((CHUNK, D), jnp.float32),
            pltpu.VMEM((CHUNK, D), jnp.float32),
            pltpu.VMEM_SHARED((N_NODES, D), jnp.float32),
            pltpu.SemaphoreType.DMA,
            pltpu.SemaphoreType.DMA,
            pltpu.SemaphoreType.DMA,
            pltpu.SemaphoreType.DMA,
            pltpu.SemaphoreType.DMA,
            pltpu.SemaphoreType.DMA,
            pltpu.SemaphoreType.DMA,
            pltpu.SemaphoreType.DMA,
            pltpu.SemaphoreType.DMA,
            pltpu.SemaphoreType.DMA,
            pltpu.SemaphoreType.DMA,
        ],
    )
    def k(src_hbm, dst_hbm, w_hbm, sup_hbm, out_hbm,
          src_r, dst_r, w_r, b0, b1, b2, o0, o1, acc,
          g0, g1, g2, s0, s1, i0, i1, i2, i3, i4, i5):
        c = lax.axis_index("c")
        s = lax.axis_index("s")
        wid = c * NS + s
        ebase = wid * CHUNKS_PER_TILE * CHUNK
        bufs = (b0, b1, b2)
        outs = (o0, o1)
        gsem = (g0, g1, g2)
        ssem = (s0, s1)
        isem = (i0, i1, i2, i3, i4, i5)

        def zbody(i, carry):
            for dd in range(D // LANES):
                b2[i, pl.ds(dd * LANES, LANES)] = jnp.zeros(
                    (LANES,), jnp.float32)
            return carry

        lax.fori_loop(0, CHUNK, zbody, 0)

        NZ = STRIPE // CHUNK
        for z in range(NZ):
            pltpu.sync_copy(b2.at[pl.ds(0, CHUNK)],
                            acc.at[pl.ds(s * STRIPE + z * CHUNK, CHUNK)])
        pltpu.sync_copy(b2.at[pl.ds(0, STRIPE - NZ * CHUNK)],
                        acc.at[pl.ds(s * STRIPE + NZ * CHUNK,
                                     STRIPE - NZ * CHUNK)])

        @pl.when(s == NS - 1)
        def _():
            pltpu.sync_copy(b2.at[pl.ds(0, TAIL)],
                            acc.at[pl.ds(NS * STRIPE, TAIL)])

        plsc.subcore_barrier()

        def idx_load(j, v):
            off = ebase + j * CHUNK
            pltpu.async_copy(src_hbm.at[pl.ds(off, CHUNK)],
                             src_r.at[v % 6], isem[v % 6])
            pltpu.async_copy(dst_hbm.at[pl.ds(off, CHUNK)],
                             dst_r.at[v % 6], isem[v % 6])
            pltpu.async_copy(w_hbm.at[pl.ds(off, CHUNK)],
                             w_r.at[v % 6], isem[v % 6])

        def wait_idx(v):
            pltpu.make_async_copy(src_hbm.at[pl.ds(0, CHUNK)],
                                  src_r.at[v % 6], isem[v % 6]).wait()
            pltpu.make_async_copy(dst_hbm.at[pl.ds(0, CHUNK)],
                                  dst_r.at[v % 6], isem[v % 6]).wait()
            pltpu.make_async_copy(w_hbm.at[pl.ds(0, CHUNK)],
                                  w_r.at[v % 6], isem[v % 6]).wait()

        def gather(v):
            pltpu.async_copy(sup_hbm.at[src_r.at[v % 6]], bufs[v % 3],
                             gsem[v % 3])

        def wait_gather(v):
            pltpu.make_async_copy(sup_hbm.at[src_r.at[v % 6]], bufs[v % 3],
                                  gsem[v % 3]).wait()

        def scale(v):
            buf = bufs[v % 3]
            out = outs[v % 2]

            def mul_body(g, inner):
                i = g * LANES
                wv = w_r[v % 6, pl.ds(i, LANES)]
                for e in range(LANES):
                    wsc = wv[e]
                    for dd in range(D // LANES):
                        sl = pl.ds(dd * LANES, LANES)
                        out[i + e, sl] = buf[i + e, sl] * wsc
                return inner

            lax.fori_loop(0, CHUNK // LANES, mul_body, 0)

        def scatter(v):
            pltpu.async_copy(outs[v % 2], acc.at[dst_r.at[v % 6]],
                             ssem[v % 2], add=True)

        def wait_scatter(v):
            pltpu.make_async_copy(outs[v % 2], acc.at[dst_r.at[v % 6]],
                                  ssem[v % 2]).wait()

        T = CHUNKS_PER_TILE // UNROLL

        idx_load(0, 0)
        idx_load(1, 1)
        idx_load(2, 2)
        idx_load(3, 3)
        wait_idx(0)
        gather(0)
        wait_idx(1)
        gather(1)

        def pipe_body(t, carry):
            j0 = t * UNROLL
            for u in range(UNROLL):
                j = j0 + u

                if u >= 2:
                    wait_scatter(u - 2)
                else:
                    @pl.when(t > 0)
                    def _(u=u):
                        wait_scatter(u + 4)

                wait_gather(u)
                scale(u)
                scatter(u)

                if u < 2:
                    idx_load(j + 4, u + 4)
                else:
                    @pl.when(t < T - 1)
                    def _(u=u, j=j):
                        idx_load(j + 4, u + 4)

                if u < UNROLL - 2:
                    wait_idx(u + 2)
                    gather(u + 2)
                else:
                    @pl.when(t < T - 1)
                    def _(u=u):
                        wait_idx(u + 2)
                        gather(u + 2)
            return carry

        lax.fori_loop(0, T, pipe_body, 0)

        wait_scatter(4)
        wait_scatter(5)
        plsc.subcore_barrier()

        pltpu.sync_copy(acc.at[pl.ds(s * STRIPE, STRIPE)],
                        out_hbm.at[c, pl.ds(s * STRIPE, STRIPE)])

        @pl.when(s == NS - 1)
        def _():
            pltpu.sync_copy(acc.at[pl.ds(NS * STRIPE, TAIL)],
                            out_hbm.at[c, pl.ds(NS * STRIPE, TAIL)])

    return k(src1, dst1, w1, support)


def kernel(x, edge_index, edge_weight, W, b):
    support = _matmul(x, W)

    src = edge_index[1].astype(jnp.int32)
    dst = edge_index[0].astype(jnp.int32)
    pad = E_PAD - N_EDGES
    zi = jnp.arange(pad, dtype=jnp.int32) % N_NODES
    src1 = jnp.concatenate([src, zi])
    dst1 = jnp.concatenate([dst, zi])
    w1 = jnp.concatenate(
        [edge_weight.astype(jnp.float32), jnp.zeros((pad,), jnp.float32)])

    partials = _sc_spmm(src1, dst1, w1, support)
    return _combine(partials, b.reshape(1, D))


# speedup vs baseline: 1.6147x; 1.6147x over previous
import functools

import jax
import jax.numpy as jnp
from jax import lax
from jax.experimental import pallas as pl
from jax.experimental.pallas import tpu as pltpu
from jax.experimental.pallas import tpu_sc as plsc

N_NODES = 10000
N_EDGES = 320000
D = 128

NC = 2
NS = 16
LANES = 16

CHUNK = 112
UNROLL = 12
CHUNKS_PER_TILE = 96
E_PAD = NC * NS * CHUNKS_PER_TILE * CHUNK
STRIPE = 624
TAIL = N_NODES - NS * STRIPE


def _matmul(x, W):
    def body(x_ref, w_ref, o_ref):
        o_ref[...] = jnp.dot(x_ref[...], w_ref[...],
                             preferred_element_type=jnp.float32)

    blk = 1000
    return pl.pallas_call(
        body,
        grid=(N_NODES // blk,),
        in_specs=[
            pl.BlockSpec((blk, D), lambda i: (i, 0)),
            pl.BlockSpec((D, D), lambda i: (0, 0)),
        ],
        out_specs=pl.BlockSpec((blk, D), lambda i: (i, 0)),
        out_shape=jax.ShapeDtypeStruct((N_NODES, D), jnp.float32),
    )(x, W)


def _combine(partials, b2):
    def body(p_ref, b_ref, o_ref):
        o_ref[...] = p_ref[0] + p_ref[1] + b_ref[...]

    blk = 1000
    return pl.pallas_call(
        body,
        grid=(N_NODES // blk,),
        in_specs=[
            pl.BlockSpec((2, blk, D), lambda i: (0, i, 0)),
            pl.BlockSpec((1, D), lambda i: (0, 0)),
        ],
        out_specs=pl.BlockSpec((blk, D), lambda i: (i, 0)),
        out_shape=jax.ShapeDtypeStruct((N_NODES, D), jnp.float32),
    )(partials, b2)


def _sc_spmm(src1, dst1, w1, support):
    mesh = plsc.VectorSubcoreMesh(core_axis_name="c", subcore_axis_name="s")

    @functools.partial(
        pl.kernel,
        out_type=jax.ShapeDtypeStruct((NC, N_NODES, D), jnp.float32),
        mesh=mesh,
        scratch_types=[
            pltpu.VMEM((4, CHUNK), jnp.int32),
            pltpu.VMEM((4, CHUNK), jnp.int32),
            pltpu.VMEM((4, CHUNK), jnp.float32),
            pltpu.VMEM((CHUNK, D), jnp.float32),
            pltpu.VMEM((CHUNK, D), jnp.float32),
            pltpu.VMEM((CHUNK, D), jnp.float32),
            pltpu.VMEM_SHARED((N_NODES, D), jnp.float32),
            pltpu.SemaphoreType.DMA,
            pltpu.SemaphoreType.DMA,
            pltpu.SemaphoreType.DMA,
            pltpu.SemaphoreType.DMA,
            pltpu.SemaphoreType.DMA,
            pltpu.SemaphoreType.DMA,
            pltpu.SemaphoreType.DMA,
            pltpu.SemaphoreType.DMA,
            pltpu.SemaphoreType.DMA,
        ],
    )
    def k(src_hbm, dst_hbm, w_hbm, sup_hbm, out_hbm,
          src_r, dst_r, w_r, b0, b1, b2, acc,
          g0, g1, g2, s0, s1, i0, i1, i2, i3):
        c = lax.axis_index("c")
        s = lax.axis_index("s")
        wid = c * NS + s
        ebase = wid * CHUNKS_PER_TILE * CHUNK
        bufs = (b0, b1, b2)
        gsem = (g0, g1, g2)
        ssem = (s0, s1)
        isem = (i0, i1, i2, i3)

        def zbody(i, carry):
            for dd in range(D // LANES):
                b2[i, pl.ds(dd * LANES, LANES)] = jnp.zeros(
                    (LANES,), jnp.float32)
            return carry

        lax.fori_loop(0, CHUNK, zbody, 0)

        NZ = STRIPE // CHUNK
        for z in range(NZ):
            pltpu.sync_copy(b2.at[pl.ds(0, CHUNK)],
                            acc.at[pl.ds(s * STRIPE + z * CHUNK, CHUNK)])
        pltpu.sync_copy(b2.at[pl.ds(0, STRIPE - NZ * CHUNK)],
                        acc.at[pl.ds(s * STRIPE + NZ * CHUNK,
                                     STRIPE - NZ * CHUNK)])

        @pl.when(s == NS - 1)
        def _():
            pltpu.sync_copy(b2.at[pl.ds(0, TAIL)],
                            acc.at[pl.ds(NS * STRIPE, TAIL)])

        plsc.subcore_barrier()

        def idx_load(j, v):
            off = ebase + j * CHUNK
            pltpu.async_copy(src_hbm.at[pl.ds(off, CHUNK)],
                             src_r.at[v % 4], isem[v % 4])
            pltpu.async_copy(dst_hbm.at[pl.ds(off, CHUNK)],
                             dst_r.at[v % 4], isem[v % 4])
            pltpu.async_copy(w_hbm.at[pl.ds(off, CHUNK)],
                             w_r.at[v % 4], isem[v % 4])

        def wait_idx(v):
            pltpu.make_async_copy(src_hbm.at[pl.ds(0, CHUNK)],
                                  src_r.at[v % 4], isem[v % 4]).wait()
            pltpu.make_async_copy(dst_hbm.at[pl.ds(0, CHUNK)],
                                  dst_r.at[v % 4], isem[v % 4]).wait()
            pltpu.make_async_copy(w_hbm.at[pl.ds(0, CHUNK)],
                                  w_r.at[v % 4], isem[v % 4]).wait()

        def gather(v):
            pltpu.async_copy(sup_hbm.at[src_r.at[v % 4]], bufs[v % 3],
                             gsem[v % 3])

        def wait_gather(v):
            pltpu.make_async_copy(sup_hbm.at[src_r.at[v % 4]], bufs[v % 3],
                                  gsem[v % 3]).wait()

        def scale(v):
            buf = bufs[v % 3]

            def mul_body(g, inner):
                i = g * LANES
                wv = w_r[v % 4, pl.ds(i, LANES)]
                for e in range(LANES):
                    wsc = wv[e]
                    for dd in range(D // LANES):
                        sl = pl.ds(dd * LANES, LANES)
                        buf[i + e, sl] = buf[i + e, sl] * wsc
                return inner

            lax.fori_loop(0, CHUNK // LANES, mul_body, 0)

        def scatter(v):
            pltpu.async_copy(bufs[v % 3], acc.at[dst_r.at[v % 4]],
                             ssem[v % 2], add=True)

        def wait_scatter(v):
            pltpu.make_async_copy(bufs[v % 3], acc.at[dst_r.at[v % 4]],
                                  ssem[v % 2]).wait()

        T = CHUNKS_PER_TILE // UNROLL

        idx_load(0, 0)
        idx_load(1, 1)
        wait_idx(0)
        gather(0)

        def pipe_body(t, carry):
            j0 = t * UNROLL
            for u in range(UNROLL):
                j = j0 + u

                if u >= 2:
                    wait_scatter(u - 2)
                else:
                    @pl.when(t > 0)
                    def _(u=u):
                        wait_scatter(u + UNROLL - 2)

                if u < UNROLL - 2:
                    idx_load(j + 2, u + 2)
                else:
                    @pl.when(t < T - 1)
                    def _(u=u, j=j):
                        idx_load(j + 2, u + 2)

                if u < UNROLL - 1:
                    wait_idx(u + 1)
                    gather(u + 1)
                else:
                    @pl.when(t < T - 1)
                    def _(u=u):
                        wait_idx(u + 1)
                        gather(u + 1)

                wait_gather(u)
                scale(u)
                scatter(u)
            return carry

        lax.fori_loop(0, T, pipe_body, 0)

        wait_scatter(UNROLL - 2)
        wait_scatter(UNROLL - 1)
        plsc.subcore_barrier()

        pltpu.sync_copy(acc.at[pl.ds(s * STRIPE, STRIPE)],
                        out_hbm.at[c, pl.ds(s * STRIPE, STRIPE)])

        @pl.when(s == NS - 1)
        def _():
            pltpu.sync_copy(acc.at[pl.ds(NS * STRIPE, TAIL)],
                            out_hbm.at[c, pl.ds(NS * STRIPE, TAIL)])

    return k(src1, dst1, w1, support)


def kernel(x, edge_index, edge_weight, W, b):
    support = _matmul(x, W)

    src = edge_index[1].astype(jnp.int32)
    dst = edge_index[0].astype(jnp.int32)
    pad = E_PAD - N_EDGES
    zi = jnp.arange(pad, dtype=jnp.int32) % N_NODES
    src1 = jnp.concatenate([src, zi])
    dst1 = jnp.concatenate([dst, zi])
    w1 = jnp.concatenate(
        [edge_weight.astype(jnp.float32), jnp.zeros((pad,), jnp.float32)])

    partials = _sc_spmm(src1, dst1, w1, support)
    return _combine(partials, b.reshape(1, D))


# speedup vs baseline: 1.6192x; 1.0028x over previous
import functools

import jax
import jax.numpy as jnp
from jax import lax
from jax.experimental import pallas as pl
from jax.experimental.pallas import tpu as pltpu
from jax.experimental.pallas import tpu_sc as plsc

N_NODES = 10000
N_EDGES = 320000
D = 128

NC = 2
NS = 16
LANES = 16

CHUNK = 112
UNROLL = 12
CHUNKS_PER_TILE = 96
E_PAD = NC * NS * CHUNKS_PER_TILE * CHUNK
STRIPE = 624
TAIL = N_NODES - NS * STRIPE


def _matmul(x, W):
    def body(x_ref, w_ref, o_ref):
        o_ref[...] = jnp.dot(x_ref[...], w_ref[...],
                             preferred_element_type=jnp.float32)

    blk = 1000
    return pl.pallas_call(
        body,
        grid=(N_NODES // blk,),
        in_specs=[
            pl.BlockSpec((blk, D), lambda i: (i, 0)),
            pl.BlockSpec((D, D), lambda i: (0, 0)),
        ],
        out_specs=pl.BlockSpec((blk, D), lambda i: (i, 0)),
        out_shape=jax.ShapeDtypeStruct((N_NODES, D), jnp.float32),
    )(x, W)


def _combine(partials, b2):
    def body(p_ref, b_ref, o_ref):
        o_ref[...] = p_ref[0] + p_ref[1] + b_ref[...]

    blk = 1000
    return pl.pallas_call(
        body,
        grid=(N_NODES // blk,),
        in_specs=[
            pl.BlockSpec((2, blk, D), lambda i: (0, i, 0)),
            pl.BlockSpec((1, D), lambda i: (0, 0)),
        ],
        out_specs=pl.BlockSpec((blk, D), lambda i: (i, 0)),
        out_shape=jax.ShapeDtypeStruct((N_NODES, D), jnp.float32),
    )(partials, b2)


def _sc_spmm(src1, dst1, w1, support):
    mesh = plsc.VectorSubcoreMesh(core_axis_name="c", subcore_axis_name="s")

    @functools.partial(
        pl.kernel,
        out_type=jax.ShapeDtypeStruct((NC, N_NODES, D), jnp.float32),
        mesh=mesh,
        scratch_types=[
            pltpu.VMEM((4, CHUNK), jnp.int32),
            pltpu.VMEM((4, CHUNK), jnp.int32),
            pltpu.VMEM((4, CHUNK), jnp.float32),
            pltpu.VMEM((CHUNK, D), jnp.float32),
            pltpu.VMEM((CHUNK, D), jnp.float32),
            pltpu.VMEM((CHUNK, D), jnp.float32),
            pltpu.VMEM_SHARED((N_NODES, D), jnp.float32),
            pltpu.SemaphoreType.DMA,
            pltpu.SemaphoreType.DMA,
            pltpu.SemaphoreType.DMA,
            pltpu.SemaphoreType.DMA,
            pltpu.SemaphoreType.DMA,
            pltpu.SemaphoreType.DMA,
            pltpu.SemaphoreType.DMA,
            pltpu.SemaphoreType.DMA,
            pltpu.SemaphoreType.DMA,
        ],
    )
    def k(src_hbm, dst_hbm, w_hbm, sup_hbm, out_hbm,
          src_r, dst_r, w_r, b0, b1, b2, acc,
          g0, g1, g2, s0, s1, i0, i1, i2, i3):
        c = lax.axis_index("c")
        s = lax.axis_index("s")
        wid = c * NS + s
        ebase = wid * CHUNKS_PER_TILE * CHUNK
        bufs = (b0, b1, b2)
        gsem = (g0, g1, g2)
        ssem = (s0, s1)
        isem = (i0, i1, i2, i3)

        def idx_load(j, v):
            off = ebase + j * CHUNK
            pltpu.async_copy(src_hbm.at[pl.ds(off, CHUNK)],
                             src_r.at[v % 4], isem[v % 4])
            pltpu.async_copy(dst_hbm.at[pl.ds(off, CHUNK)],
                             dst_r.at[v % 4], isem[v % 4])
            pltpu.async_copy(w_hbm.at[pl.ds(off, CHUNK)],
                             w_r.at[v % 4], isem[v % 4])

        def wait_idx(v):
            pltpu.make_async_copy(src_hbm.at[pl.ds(0, CHUNK)],
                                  src_r.at[v % 4], isem[v % 4]).wait()
            pltpu.make_async_copy(dst_hbm.at[pl.ds(0, CHUNK)],
                                  dst_r.at[v % 4], isem[v % 4]).wait()
            pltpu.make_async_copy(w_hbm.at[pl.ds(0, CHUNK)],
                                  w_r.at[v % 4], isem[v % 4]).wait()

        def gather(v):
            pltpu.async_copy(sup_hbm.at[src_r.at[v % 4]], bufs[v % 3],
                             gsem[v % 3])

        def wait_gather(v):
            pltpu.make_async_copy(sup_hbm.at[src_r.at[v % 4]], bufs[v % 3],
                                  gsem[v % 3]).wait()

        def scale(v):
            buf = bufs[v % 3]

            def mul_body(g, inner):
                i = g * LANES
                wv = w_r[v % 4, pl.ds(i, LANES)]
                for e in range(LANES):
                    wsc = wv[e]
                    for dd in range(D // LANES):
                        sl = pl.ds(dd * LANES, LANES)
                        buf[i + e, sl] = buf[i + e, sl] * wsc
                return inner

            lax.fori_loop(0, CHUNK // LANES, mul_body, 0)

        def scatter(v):
            pltpu.async_copy(bufs[v % 3], acc.at[dst_r.at[v % 4]],
                             ssem[v % 2], add=True)

        def wait_scatter(v):
            pltpu.make_async_copy(bufs[v % 3], acc.at[dst_r.at[v % 4]],
                                  ssem[v % 2]).wait()

        T = CHUNKS_PER_TILE // UNROLL

        idx_load(0, 0)
        idx_load(1, 1)
        wait_idx(0)
        gather(0)

        def zbody(i, carry):
            for dd in range(D // LANES):
                b2[i, pl.ds(dd * LANES, LANES)] = jnp.zeros(
                    (LANES,), jnp.float32)
            return carry

        lax.fori_loop(0, CHUNK, zbody, 0)

        NZ = STRIPE // CHUNK
        for z in range(NZ):
            pltpu.sync_copy(b2.at[pl.ds(0, CHUNK)],
                            acc.at[pl.ds(s * STRIPE + z * CHUNK, CHUNK)])
        pltpu.sync_copy(b2.at[pl.ds(0, STRIPE - NZ * CHUNK)],
                        acc.at[pl.ds(s * STRIPE + NZ * CHUNK,
                                     STRIPE - NZ * CHUNK)])

        @pl.when(s == NS - 1)
        def _():
            pltpu.sync_copy(b2.at[pl.ds(0, TAIL)],
                            acc.at[pl.ds(NS * STRIPE, TAIL)])

        plsc.subcore_barrier()

        def pipe_body(t, carry):
            j0 = t * UNROLL
            for u in range(UNROLL):
                j = j0 + u

                if u >= 2:
                    wait_scatter(u - 2)
                else:
                    @pl.when(t > 0)
                    def _(u=u):
                        wait_scatter(u + UNROLL - 2)

                if u < UNROLL - 2:
                    idx_load(j + 2, u + 2)
                else:
                    @pl.when(t < T - 1)
                    def _(u=u, j=j):
                        idx_load(j + 2, u + 2)

                if u < UNROLL - 1:
                    wait_idx(u + 1)
                    gather(u + 1)
                else:
                    @pl.when(t < T - 1)
                    def _(u=u):
                        wait_idx(u + 1)
                        gather(u + 1)

                wait_gather(u)
                scale(u)
                scatter(u)
            return carry

        lax.fori_loop(0, T, pipe_body, 0)

        wait_scatter(UNROLL - 2)
        wait_scatter(UNROLL - 1)
        plsc.subcore_barrier()

        pltpu.sync_copy(acc.at[pl.ds(s * STRIPE, STRIPE)],
                        out_hbm.at[c, pl.ds(s * STRIPE, STRIPE)])

        @pl.when(s == NS - 1)
        def _():
            pltpu.sync_copy(acc.at[pl.ds(NS * STRIPE, TAIL)],
                            out_hbm.at[c, pl.ds(NS * STRIPE, TAIL)])

    return k(src1, dst1, w1, support)


def kernel(x, edge_index, edge_weight, W, b):
    support = _matmul(x, W)

    src = edge_index[1].astype(jnp.int32)
    dst = edge_index[0].astype(jnp.int32)
    pad = E_PAD - N_EDGES
    zi = jnp.arange(pad, dtype=jnp.int32) % N_NODES
    src1 = jnp.concatenate([src, zi])
    dst1 = jnp.concatenate([dst, zi])
    w1 = jnp.concatenate(
        [edge_weight.astype(jnp.float32), jnp.zeros((pad,), jnp.float32)])

    partials = _sc_spmm(src1, dst1, w1, support)
    return _combine(partials, b.reshape(1, D))
